# Initial kernel scaffold; baseline (speedup 1.0000x reference)
import functools

import jax
import jax.numpy as jnp
from jax import lax
from jax.experimental import pallas as pl
from jax.experimental.pallas import tpu as pltpu
from jax.experimental.pallas import tpu_sc as plsc

D = 64
LANES = 16
FIRE = 128
FIRES_PER_CHUNK = 5
CHUNK = FIRE * FIRES_PER_CHUNK


def _make_encoder(total_b):
    info = plsc.get_sparse_core_info()
    nc, ns = info.num_cores, info.num_subcores
    nw = nc * ns
    per_w = total_b // nw
    assert total_b % nw == 0 and per_w % CHUNK == 0
    n_chunks = per_w // CHUNK

    mesh = plsc.VectorSubcoreMesh(core_axis_name="c", subcore_axis_name="s")

    @functools.partial(
        pl.kernel,
        mesh=mesh,
        out_type=jax.ShapeDtypeStruct((total_b, D), jnp.float32),
        scratch_types=[
            pltpu.VMEM((per_w,), jnp.int32),
            pltpu.VMEM((per_w,), jnp.int32),
            pltpu.VMEM((CHUNK, D), jnp.float32),
            pltpu.VMEM((CHUNK, D), jnp.float32),
            pltpu.SemaphoreType.DMA,
            pltpu.SemaphoreType.DMA,
        ],
    )
    def enc(node_hbm, deg_hbm, nidx_hbm, didx_hbm, out_hbm,
            nidx_v, didx_v, nrows_v, drows_v, nsem, dsem):
        wid = lax.axis_index("s") * nc + lax.axis_index("c")
        base = wid * per_w
        pltpu.sync_copy(nidx_hbm.at[pl.ds(base, per_w)], nidx_v)
        pltpu.sync_copy(didx_hbm.at[pl.ds(base, per_w)], didx_v)

        def chunk_body(ci, carry):
            off = ci * CHUNK
            copies = []
            for f in range(FIRES_PER_CHUNK):
                fo = off + f * FIRE
                copies.append(pltpu.async_copy(
                    node_hbm.at[nidx_v.at[pl.ds(fo, FIRE)]],
                    nrows_v.at[pl.ds(f * FIRE, FIRE)], nsem))
                copies.append(pltpu.async_copy(
                    deg_hbm.at[didx_v.at[pl.ds(fo, FIRE)]],
                    drows_v.at[pl.ds(f * FIRE, FIRE)], dsem))
            for cp in copies:
                cp.wait()

            def add_body(r, c2):
                for g in range(D // LANES):
                    sl = pl.ds(g * LANES, LANES)
                    nrows_v[r, sl] = nrows_v[r, sl] + drows_v[r, sl]
                return c2

            lax.fori_loop(0, CHUNK, add_body, 0)
            pltpu.sync_copy(nrows_v, out_hbm.at[pl.ds(base + off, CHUNK)])
            return carry

        lax.fori_loop(0, n_chunks, chunk_body, 0)

    return enc


@jax.jit
def kernel(user_seq, user_degree, node_table, degree_table):
    b, s = user_seq.shape
    enc = _make_encoder(b * s)
    out = enc(node_table, degree_table,
              user_seq.reshape(-1), user_degree.reshape(-1))
    return out.reshape(b, s, D)


# speedup vs baseline: 6.3791x; 6.3791x over previous
import functools

import jax
import jax.numpy as jnp
from jax import lax
from jax.experimental import pallas as pl
from jax.experimental.pallas import tpu as pltpu
from jax.experimental.pallas import tpu_sc as plsc

D = 64
LANES = 16
FIRE = 128
FIRES_PER_CHUNK = 5
CHUNK = FIRE * FIRES_PER_CHUNK


def _make_encoder(total_b):
    info = plsc.get_sparse_core_info()
    nc, ns = info.num_cores, info.num_subcores
    nw = nc * ns
    per_w = total_b // nw
    assert total_b % nw == 0 and per_w % CHUNK == 0
    n_chunks = per_w // CHUNK

    mesh = plsc.VectorSubcoreMesh(core_axis_name="c", subcore_axis_name="s")

    @functools.partial(
        pl.kernel,
        mesh=mesh,
        compiler_params=pltpu.CompilerParams(use_tc_tiling_on_sc=False),
        out_type=jax.ShapeDtypeStruct((total_b, D), jnp.float32),
        scratch_types=[
            pltpu.VMEM((per_w,), jnp.int32),
            pltpu.VMEM((per_w,), jnp.int32),
            pltpu.VMEM((CHUNK, D), jnp.float32),
            pltpu.VMEM((CHUNK, D), jnp.float32),
            pltpu.SemaphoreType.DMA,
            pltpu.SemaphoreType.DMA,
        ],
    )
    def enc(node_hbm, deg_hbm, nidx_hbm, didx_hbm, out_hbm,
            nidx_v, didx_v, nrows_v, drows_v, nsem, dsem):
        wid = lax.axis_index("s") * nc + lax.axis_index("c")
        base = wid * per_w
        pltpu.sync_copy(nidx_hbm.at[pl.ds(base, per_w)], nidx_v)
        pltpu.sync_copy(didx_hbm.at[pl.ds(base, per_w)], didx_v)

        def chunk_body(ci, carry):
            off = ci * CHUNK
            copies = []
            for f in range(FIRES_PER_CHUNK):
                fo = off + f * FIRE
                copies.append(pltpu.async_copy(
                    node_hbm.at[nidx_v.at[pl.ds(fo, FIRE)]],
                    nrows_v.at[pl.ds(f * FIRE, FIRE)], nsem))
                copies.append(pltpu.async_copy(
                    deg_hbm.at[didx_v.at[pl.ds(fo, FIRE)]],
                    drows_v.at[pl.ds(f * FIRE, FIRE)], dsem))
            for cp in copies:
                cp.wait()

            def add_body(r, c2):
                for g in range(D // LANES):
                    sl = pl.ds(g * LANES, LANES)
                    nrows_v[r, sl] = nrows_v[r, sl] + drows_v[r, sl]
                return c2

            lax.fori_loop(0, CHUNK, add_body, 0)
            pltpu.sync_copy(nrows_v, out_hbm.at[pl.ds(base + off, CHUNK)])
            return carry

        lax.fori_loop(0, n_chunks, chunk_body, 0)

    return enc


@jax.jit
def kernel(user_seq, user_degree, node_table, degree_table):
    b, s = user_seq.shape
    enc = _make_encoder(b * s)
    out = enc(node_table, degree_table,
              user_seq.reshape(-1), user_degree.reshape(-1))
    return out.reshape(b, s, D)


# speedup vs baseline: 6.5064x; 1.0199x over previous
import functools

import jax
import jax.numpy as jnp
from jax import lax
from jax.experimental import pallas as pl
from jax.experimental.pallas import tpu as pltpu
from jax.experimental.pallas import tpu_sc as plsc

D = 64
LANES = 16
FIRE = 128
FIRES_PER_CHUNK = 5
CHUNK = FIRE * FIRES_PER_CHUNK


def _make_encoder(total_b):
    info = plsc.get_sparse_core_info()
    nc, ns = info.num_cores, info.num_subcores
    nw = nc * ns
    per_w = total_b // nw
    assert total_b % nw == 0 and per_w % CHUNK == 0
    n_chunks = per_w // CHUNK

    mesh = plsc.VectorSubcoreMesh(core_axis_name="c", subcore_axis_name="s")

    @functools.partial(
        pl.kernel,
        mesh=mesh,
        compiler_params=pltpu.CompilerParams(use_tc_tiling_on_sc=False),
        out_type=jax.ShapeDtypeStruct((total_b, D), jnp.float32),
        scratch_types=[
            pltpu.VMEM((per_w,), jnp.int32),
            pltpu.VMEM((per_w,), jnp.int32),
            pltpu.VMEM((CHUNK, D), jnp.float32),
            pltpu.VMEM((CHUNK, D), jnp.float32),
            pltpu.SemaphoreType.DMA,
            pltpu.SemaphoreType.DMA,
        ],
    )
    def enc(node_hbm, deg_hbm, nidx_hbm, didx_hbm, out_hbm,
            nidx_v, didx_v, nrows_v, drows_v, nsem, dsem):
        wid = lax.axis_index("s") * nc + lax.axis_index("c")
        base = wid * per_w
        pltpu.sync_copy(nidx_hbm.at[pl.ds(base, per_w)], nidx_v)
        pltpu.sync_copy(didx_hbm.at[pl.ds(base, per_w)], didx_v)

        def chunk_body(ci, carry):
            off = ci * CHUNK
            copies = []
            for f in range(FIRES_PER_CHUNK):
                fo = off + f * FIRE
                copies.append(pltpu.async_copy(
                    node_hbm.at[nidx_v.at[pl.ds(fo, FIRE)]],
                    nrows_v.at[pl.ds(f * FIRE, FIRE)], nsem))
            for cp in copies:
                cp.wait()
            copies = []
            for f in range(FIRES_PER_CHUNK):
                fo = off + f * FIRE
                copies.append(pltpu.async_copy(
                    deg_hbm.at[didx_v.at[pl.ds(fo, FIRE)]],
                    nrows_v.at[pl.ds(f * FIRE, FIRE)], dsem, add=True))
            for cp in copies:
                cp.wait()
            pltpu.sync_copy(nrows_v, out_hbm.at[pl.ds(base + off, CHUNK)])
            return carry

        lax.fori_loop(0, n_chunks, chunk_body, 0)

    return enc


@jax.jit
def kernel(user_seq, user_degree, node_table, degree_table):
    b, s = user_seq.shape
    enc = _make_encoder(b * s)
    out = enc(node_table, degree_table,
              user_seq.reshape(-1), user_degree.reshape(-1))
    return out.reshape(b, s, D)
